# baseline (speedup 1.0000x reference)
import functools

import jax
import jax.numpy as jnp
from jax import lax
from jax.experimental import pallas as pl
from jax.experimental.pallas import tpu as pltpu
from jax.experimental.pallas import tpu_sc as plsc

N = 10000
E = 320000
D = 128
G = 100
PER = 100
K = 30

NC = 2
NS = 16
NW = NC * NS
NPAD = 10240
NPW = NPAD // NW
CH = 128
EPAD = E + 2 * CH

_RB = 1000


@functools.cache
def _mesh():
  return plsc.VectorSubcoreMesh(core_axis_name="c", subcore_axis_name="s",
                                num_cores=NC, num_subcores=NS)


@functools.cache
def _make_fold(F):
  NV = F // 16

  @functools.partial(
      pl.kernel,
      out_type=jax.ShapeDtypeStruct((NPAD, F), jnp.float32),
      mesh=_mesh(),
      compiler_params=pltpu.CompilerParams(use_tc_tiling_on_sc=False),
      scratch_types=[
          pltpu.VMEM((CH,), jnp.int32),
          pltpu.VMEM((CH,), jnp.int32),
          pltpu.VMEM((CH, F), jnp.float32),
          pltpu.VMEM((NPW + 8, F), jnp.float32),
          pltpu.VMEM((336,), jnp.int32),
          pltpu.SemaphoreType.DMA,
      ],
  )
  def fold(cur_hbm, srcs_hbm, fdl_hbm, rpw_hbm, out_hbm, idx_v, fdl_v, rows_v,
           out_v, rp_v, sem):
    c = lax.axis_index("c")
    s = lax.axis_index("s")
    w = s * NC + c
    pltpu.sync_copy(rpw_hbm.at[w], rp_v)
    e0 = rp_v[pl.ds(0, 16)][0]
    e1 = rp_v[pl.ds(NPW, 16)][0]
    base0 = (e0 // 8) * 8
    nch = (e1 - base0 + (CH - 1)) // CH

    zero = jnp.zeros((16,), jnp.float32)

    def zbody(n, carry):
      for k in range(NV):
        out_v[n, pl.ds(16 * k, 16)] = zero
      return carry

    lax.fori_loop(0, NPW, zbody, 0)

    def chunk(ci, vs):
      base = base0 + ci * CH
      pltpu.sync_copy(srcs_hbm.at[pl.ds(base, CH)], idx_v)
      pltpu.sync_copy(fdl_hbm.at[pl.ds(base, CH)], fdl_v)
      pltpu.async_copy(cur_hbm.at[idx_v], rows_v, sem).wait()

      def block(bi, vs):
        fv = fdl_v[pl.ds(16 * bi, 16)]
        for j in range(16):
          fdl = fv[j]
          flag = fdl >= 512
          dloc = lax.rem(fdl, 512)
          g = base + 16 * bi + j
          i = 16 * bi + j
          in_range = jnp.logical_and(g >= e0, g < e1)
          vs2 = tuple(vs[k] + rows_v[i, pl.ds(16 * k, 16)]
                      for k in range(NV))
          if NV <= 2:
            dloc_eff = jnp.where(in_range, dloc, jnp.int32(NPW))
            for k in range(NV):
              out_v[dloc_eff, pl.ds(16 * k, 16)] = vs2[k]
          else:
            do_store = jnp.logical_and(flag, in_range)

            @pl.when(do_store)
            def _(vs2=vs2, dloc=dloc):
              for k in range(NV):
                out_v[dloc, pl.ds(16 * k, 16)] = vs2[k]

          vs = tuple(jnp.where(flag, zero, v2) for v2 in vs2)
        return vs

      return lax.fori_loop(0, CH // 16, block, vs)

    vs0 = tuple(jnp.zeros((16,), jnp.float32) for _ in range(NV))
    lax.fori_loop(0, nch, chunk, vs0)
    pltpu.sync_copy(out_v.at[pl.ds(0, NPW)], out_hbm.at[pl.ds(w * NPW, NPW)])

  return fold


_BPW = 104


@functools.cache
def _make_gather_rows():
  @functools.partial(
      pl.kernel,
      out_type=jax.ShapeDtypeStruct((_BPW * NW, 112), jnp.float32),
      mesh=_mesh(),
      compiler_params=pltpu.CompilerParams(use_tc_tiling_on_sc=False),
      scratch_types=[
          pltpu.VMEM((_BPW,), jnp.int32),
          pltpu.VMEM((_BPW, 112), jnp.float32),
          pltpu.SemaphoreType.DMA,
      ],
  )
  def gather(tab_hbm, idx_hbm, out_hbm, idx_v, rows_v, sem):
    c = lax.axis_index("c")
    s = lax.axis_index("s")
    base = (s * NC + c) * _BPW
    pltpu.sync_copy(idx_hbm.at[pl.ds(base, _BPW)], idx_v)
    pltpu.async_copy(tab_hbm.at[idx_v], rows_v, sem).wait()
    pltpu.sync_copy(rows_v, out_hbm.at[pl.ds(base, _BPW)])

  return gather


def _layer(seg, cur, b, deg, wn):
  fi = cur.shape[1]
  fo = wn.shape[1]

  def body(seg_ref, cur_ref, b_ref, deg_ref, w_ref, o_ref):
    pool = seg_ref[...] + cur_ref[...]
    lin = jnp.dot(pool, w_ref[...], preferred_element_type=jnp.float32)
    o_ref[...] = jnp.tanh((lin + b_ref[...]) / deg_ref[...])

  return pl.pallas_call(
      body,
      grid=(N // _RB,),
      in_specs=[
          pl.BlockSpec((_RB, fi), lambda i: (i, 0)),
          pl.BlockSpec((_RB, fi), lambda i: (i, 0)),
          pl.BlockSpec((1, fo), lambda i: (0, 0)),
          pl.BlockSpec((_RB, 1), lambda i: (i, 0)),
          pl.BlockSpec((fi, fo), lambda i: (0, 0)),
      ],
      out_specs=pl.BlockSpec((_RB, fo), lambda i: (i, 0)),
      out_shape=jax.ShapeDtypeStruct((N, fo), jnp.float32),
  )(seg, cur, b, deg, wn)


def _topk(spad):
  def body(x_ref, o_ref):
    v = x_ref[...]
    lanes = lax.broadcasted_iota(jnp.int32, (104, 128), 1)
    gcol = lax.broadcasted_iota(jnp.int32, (104, 1), 0)
    for k in range(32):
      m = jnp.max(v, axis=1, keepdims=True)
      cand = jnp.where(v == m, lanes, jnp.int32(1 << 30))
      idx = jnp.min(cand, axis=1, keepdims=True)
      o_ref[:, k:k + 1] = jnp.minimum(idx + gcol * PER, N - 1)
      v = jnp.where(lanes == idx, jnp.float32(-3.0), v)

  return pl.pallas_call(
      body,
      in_specs=[pl.BlockSpec((104, 128), lambda: (0, 0))],
      out_specs=pl.BlockSpec((104, 32), lambda: (0, 0)),
      out_shape=jax.ShapeDtypeStruct((104, 32), jnp.int32),
  )(spad)


def _convs(pooled3, w1, b1, w2s, b2, sel):
  def body(p_ref, w1_ref, b1_ref, w2_ref, b2_ref, sel_ref, o_ref):
    p = p_ref[0]
    m1 = jnp.dot(p, w1_ref[...], preferred_element_type=jnp.float32)
    m1 = jnp.maximum(m1 + b1_ref[...], 0.0)
    me = jnp.dot(sel_ref[0], m1, preferred_element_type=jnp.float32)
    mo = jnp.dot(sel_ref[1], m1, preferred_element_type=jnp.float32)
    m = jnp.maximum(me, mo)
    acc = jnp.dot(m[0:11, :], w2_ref[0],
                  preferred_element_type=jnp.float32)
    for dk in range(1, 5):
      acc = acc + jnp.dot(m[dk:dk + 11, :], w2_ref[dk],
                          preferred_element_type=jnp.float32)
    o_ref[0] = jnp.maximum(acc + b2_ref[...], 0.0)

  return pl.pallas_call(
      body,
      grid=(G,),
      in_specs=[
          pl.BlockSpec((1, 32, 112), lambda i: (i, 0, 0)),
          pl.BlockSpec((112, 16), lambda i: (0, 0)),
          pl.BlockSpec((1, 16), lambda i: (0, 0)),
          pl.BlockSpec((5, 16, 32), lambda i: (0, 0, 0)),
          pl.BlockSpec((1, 32), lambda i: (0, 0)),
          pl.BlockSpec((2, 16, 32), lambda i: (0, 0, 0)),
      ],
      out_specs=pl.BlockSpec((1, 11, 32), lambda i: (i, 0, 0)),
      out_shape=jax.ShapeDtypeStruct((G, 11, 32), jnp.float32),
  )(pooled3, w1, b1, w2s, b2, sel)


def kernel(node_feat, edge_index, W0, b0, W1, b1, W2, b2, W3, b3, conv1_w,
           conv1_b, conv2_w, conv2_b):
  src = edge_index[0]
  dst = edge_index[1]

  order = jnp.argsort(dst, stable=True)
  srcs = src[order]
  dsts = dst[order]
  rowptr = jnp.searchsorted(
      dsts, jnp.arange(NPAD + 1, dtype=jnp.int32), side="left"
  ).astype(jnp.int32)
  deg = (rowptr[1:N + 1] - rowptr[:N]).astype(jnp.float32)[:, None] + 1.0

  flag = jnp.concatenate(
      [dsts[1:] != dsts[:-1], jnp.ones((1,), jnp.bool_)]).astype(jnp.int32)
  fdl = lax.rem(dsts, jnp.int32(NPW)) + 512 * flag
  srcs_p = jnp.concatenate([srcs, jnp.zeros((EPAD - E,), jnp.int32)])
  fdl_p = jnp.concatenate([fdl, jnp.zeros((EPAD - E,), jnp.int32)])
  rpw = rowptr[jnp.arange(NW)[:, None] * NPW +
               jnp.arange(NPW + 1)[None, :]]
  rpw = jnp.pad(rpw, ((0, 0), (0, 336 - (NPW + 1))))

  fold128 = _make_fold(128)
  fold32 = _make_fold(32)

  s0 = fold128(node_feat, srcs_p, fdl_p, rpw)
  cur0 = _layer(s0, node_feat, b0[None], deg, W0)
  s1 = fold32(cur0, srcs_p, fdl_p, rpw)
  cur1 = _layer(s1, cur0, b1[None], deg, W1)
  s2 = fold32(cur1, srcs_p, fdl_p, rpw)
  cur2 = _layer(s2, cur1, b2[None], deg, W2)
  s3 = fold32(cur2, srcs_p, fdl_p, rpw)
  cur3 = _layer(s3, cur2, b3[None], deg, W3)

  sortc = cur3[:, 0].reshape(G, PER)
  spad = jnp.pad(sortc, ((0, 4), (0, 28)), constant_values=-1e30)
  gidx = _topk(spad)
  global_idx = gidx[:G, :K]

  cats = jnp.concatenate(
      [cur0, cur1, cur2, cur3, jnp.zeros((N, 15), jnp.float32)], axis=1)
  pooled = _make_gather_rows()(cats, gidx.reshape(-1))
  pooled3 = pooled.reshape(104, 32, 112)[:G]

  w1eff = jnp.pad(conv1_w[:, 0, :].T, ((0, 15), (0, 0)))
  w2s = jnp.transpose(conv2_w, (2, 1, 0))
  eye = jnp.eye(16, dtype=jnp.float32)
  sel = jnp.stack([
      jnp.kron(eye, jnp.array([[1.0, 0.0]], jnp.float32)),
      jnp.kron(eye, jnp.array([[0.0, 1.0]], jnp.float32)),
  ])

  z = _convs(pooled3, w1eff, conv1_b[None], w2s, conv2_b[None], sel)
  out = z.transpose(0, 2, 1).reshape(G, -1)
  return (out, global_idx)


# speedup vs baseline: 1.1240x; 1.1240x over previous
import functools

import jax
import jax.numpy as jnp
from jax import lax
from jax.experimental import pallas as pl
from jax.experimental.pallas import tpu as pltpu
from jax.experimental.pallas import tpu_sc as plsc

N = 10000
E = 320000
D = 128
G = 100
PER = 100
K = 30

NC = 2
NS = 16
NW = NC * NS
NPAD = 10240
NPW = NPAD // NW
CH = 128
EPAD = E + 2 * CH

_RB = 1000


@functools.cache
def _mesh():
  return plsc.VectorSubcoreMesh(core_axis_name="c", subcore_axis_name="s",
                                num_cores=NC, num_subcores=NS)


@functools.cache
def _make_fold(F):
  NV = F // 16

  @functools.partial(
      pl.kernel,
      out_type=jax.ShapeDtypeStruct((NPAD, F), jnp.float32),
      mesh=_mesh(),
      compiler_params=pltpu.CompilerParams(use_tc_tiling_on_sc=False),
      scratch_types=[
          pltpu.VMEM((2, CH), jnp.int32),
          pltpu.VMEM((2, CH), jnp.int32),
          pltpu.VMEM((2, CH, F), jnp.float32),
          pltpu.VMEM((NPW + 8, F), jnp.float32),
          pltpu.VMEM((336,), jnp.int32),
          pltpu.SemaphoreType.DMA,
          pltpu.SemaphoreType.DMA,
      ],
  )
  def fold(cur_hbm, srcs_hbm, fdl_hbm, rpw_hbm, out_hbm, idx_v, fdl_v, rows_v,
           out_v, rp_v, sem0, sem1):
    c = lax.axis_index("c")
    s = lax.axis_index("s")
    w = s * NC + c
    pltpu.sync_copy(rpw_hbm.at[w], rp_v)
    e0 = rp_v[pl.ds(0, 16)][0]
    e1 = rp_v[pl.ds(NPW, 16)][0]
    base0 = (e0 // 8) * 8
    nch = (e1 - base0 + (CH - 1)) // CH

    zero = jnp.zeros((16,), jnp.float32)

    def zbody(n, carry):
      for k in range(NV):
        out_v[n, pl.ds(16 * k, 16)] = zero
      return carry

    lax.fori_loop(0, NPW, zbody, 0)

    def start(ci, p, sem):
      base = base0 + ci * CH
      pltpu.sync_copy(srcs_hbm.at[pl.ds(base, CH)], idx_v.at[p])
      pltpu.sync_copy(fdl_hbm.at[pl.ds(base, CH)], fdl_v.at[p])
      pltpu.async_copy(cur_hbm.at[idx_v.at[p]], rows_v.at[p], sem)

    def fold_chunk(ci, p, sem, vs):
      pltpu.make_async_copy(cur_hbm.at[idx_v.at[p]], rows_v.at[p], sem).wait()
      base = base0 + ci * CH

      def block(bi, vs):
        fv = fdl_v[p, pl.ds(16 * bi, 16)]
        for j in range(16):
          fdl = fv[j]
          flag = fdl >= 512
          dloc = lax.rem(fdl, 512)
          g = base + 16 * bi + j
          i = 16 * bi + j
          in_range = jnp.logical_and(g >= e0, g < e1)
          vs2 = tuple(vs[k] + rows_v[p, i, pl.ds(16 * k, 16)]
                      for k in range(NV))
          if NV <= 2:
            dloc_eff = jnp.where(in_range, dloc, jnp.int32(NPW))
            for k in range(NV):
              out_v[dloc_eff, pl.ds(16 * k, 16)] = vs2[k]
          else:
            do_store = jnp.logical_and(flag, in_range)

            @pl.when(do_store)
            def _(vs2=vs2, dloc=dloc):
              for k in range(NV):
                out_v[dloc, pl.ds(16 * k, 16)] = vs2[k]

          vs = tuple(jnp.where(flag, zero, v2) for v2 in vs2)
        return vs

      return lax.fori_loop(0, CH // 16, block, vs)

    @pl.when(nch > 0)
    def _():
      start(0, 0, sem0)

    def pair(cp, vs):
      c0 = 2 * cp
      c1 = c0 + 1
      start(c1, 1, sem1)
      vs = fold_chunk(c0, 0, sem0, vs)

      @pl.when(c0 + 2 < nch)
      def _():
        start(c0 + 2, 0, sem0)

      return fold_chunk(c1, 1, sem1, vs)

    vs0 = tuple(jnp.zeros((16,), jnp.float32) for _ in range(NV))
    vsf = lax.fori_loop(0, nch // 2, pair, vs0)

    @pl.when(lax.rem(nch, 2) == 1)
    def _():
      fold_chunk(nch - 1, 0, sem0, vsf)
    pltpu.sync_copy(out_v.at[pl.ds(0, NPW)], out_hbm.at[pl.ds(w * NPW, NPW)])

  return fold


_BPW = 104


@functools.cache
def _make_gather_rows():
  @functools.partial(
      pl.kernel,
      out_type=jax.ShapeDtypeStruct((_BPW * NW, 112), jnp.float32),
      mesh=_mesh(),
      compiler_params=pltpu.CompilerParams(use_tc_tiling_on_sc=False),
      scratch_types=[
          pltpu.VMEM((_BPW,), jnp.int32),
          pltpu.VMEM((_BPW, 112), jnp.float32),
          pltpu.SemaphoreType.DMA,
      ],
  )
  def gather(tab_hbm, idx_hbm, out_hbm, idx_v, rows_v, sem):
    c = lax.axis_index("c")
    s = lax.axis_index("s")
    base = (s * NC + c) * _BPW
    pltpu.sync_copy(idx_hbm.at[pl.ds(base, _BPW)], idx_v)
    pltpu.async_copy(tab_hbm.at[idx_v], rows_v, sem).wait()
    pltpu.sync_copy(rows_v, out_hbm.at[pl.ds(base, _BPW)])

  return gather


def _layer(seg, cur, b, deg, wn):
  fi = cur.shape[1]
  fo = wn.shape[1]

  def body(seg_ref, cur_ref, b_ref, deg_ref, w_ref, o_ref):
    pool = seg_ref[...] + cur_ref[...]
    lin = jnp.dot(pool, w_ref[...], preferred_element_type=jnp.float32)
    o_ref[...] = jnp.tanh((lin + b_ref[...]) / deg_ref[...])

  return pl.pallas_call(
      body,
      grid=(N // _RB,),
      in_specs=[
          pl.BlockSpec((_RB, fi), lambda i: (i, 0)),
          pl.BlockSpec((_RB, fi), lambda i: (i, 0)),
          pl.BlockSpec((1, fo), lambda i: (0, 0)),
          pl.BlockSpec((_RB, 1), lambda i: (i, 0)),
          pl.BlockSpec((fi, fo), lambda i: (0, 0)),
      ],
      out_specs=pl.BlockSpec((_RB, fo), lambda i: (i, 0)),
      out_shape=jax.ShapeDtypeStruct((N, fo), jnp.float32),
  )(seg, cur, b, deg, wn)


def _topk(spad):
  def body(x_ref, o_ref):
    v = x_ref[...]
    lanes = lax.broadcasted_iota(jnp.int32, (104, 128), 1)
    gcol = lax.broadcasted_iota(jnp.int32, (104, 1), 0)
    for k in range(32):
      m = jnp.max(v, axis=1, keepdims=True)
      cand = jnp.where(v == m, lanes, jnp.int32(1 << 30))
      idx = jnp.min(cand, axis=1, keepdims=True)
      o_ref[:, k:k + 1] = jnp.minimum(idx + gcol * PER, N - 1)
      v = jnp.where(lanes == idx, jnp.float32(-3.0), v)

  return pl.pallas_call(
      body,
      in_specs=[pl.BlockSpec((104, 128), lambda: (0, 0))],
      out_specs=pl.BlockSpec((104, 32), lambda: (0, 0)),
      out_shape=jax.ShapeDtypeStruct((104, 32), jnp.int32),
  )(spad)


def _convs(pooled3, w1, b1, w2s, b2, sel):
  def body(p_ref, w1_ref, b1_ref, w2_ref, b2_ref, sel_ref, o_ref):
    p = p_ref[0]
    m1 = jnp.dot(p, w1_ref[...], preferred_element_type=jnp.float32)
    m1 = jnp.maximum(m1 + b1_ref[...], 0.0)
    me = jnp.dot(sel_ref[0], m1, preferred_element_type=jnp.float32)
    mo = jnp.dot(sel_ref[1], m1, preferred_element_type=jnp.float32)
    m = jnp.maximum(me, mo)
    acc = jnp.dot(m[0:11, :], w2_ref[0],
                  preferred_element_type=jnp.float32)
    for dk in range(1, 5):
      acc = acc + jnp.dot(m[dk:dk + 11, :], w2_ref[dk],
                          preferred_element_type=jnp.float32)
    o_ref[0] = jnp.maximum(acc + b2_ref[...], 0.0)

  return pl.pallas_call(
      body,
      grid=(G,),
      in_specs=[
          pl.BlockSpec((1, 32, 112), lambda i: (i, 0, 0)),
          pl.BlockSpec((112, 16), lambda i: (0, 0)),
          pl.BlockSpec((1, 16), lambda i: (0, 0)),
          pl.BlockSpec((5, 16, 32), lambda i: (0, 0, 0)),
          pl.BlockSpec((1, 32), lambda i: (0, 0)),
          pl.BlockSpec((2, 16, 32), lambda i: (0, 0, 0)),
      ],
      out_specs=pl.BlockSpec((1, 11, 32), lambda i: (i, 0, 0)),
      out_shape=jax.ShapeDtypeStruct((G, 11, 32), jnp.float32),
  )(pooled3, w1, b1, w2s, b2, sel)


def kernel(node_feat, edge_index, W0, b0, W1, b1, W2, b2, W3, b3, conv1_w,
           conv1_b, conv2_w, conv2_b):
  src = edge_index[0]
  dst = edge_index[1]

  order = jnp.argsort(dst, stable=True)
  srcs = src[order]
  dsts = dst[order]
  rowptr = jnp.searchsorted(
      dsts, jnp.arange(NPAD + 1, dtype=jnp.int32), side="left"
  ).astype(jnp.int32)
  deg = (rowptr[1:N + 1] - rowptr[:N]).astype(jnp.float32)[:, None] + 1.0

  flag = jnp.concatenate(
      [dsts[1:] != dsts[:-1], jnp.ones((1,), jnp.bool_)]).astype(jnp.int32)
  fdl = lax.rem(dsts, jnp.int32(NPW)) + 512 * flag
  srcs_p = jnp.concatenate([srcs, jnp.zeros((EPAD - E,), jnp.int32)])
  fdl_p = jnp.concatenate([fdl, jnp.zeros((EPAD - E,), jnp.int32)])
  rpw = rowptr[jnp.arange(NW)[:, None] * NPW +
               jnp.arange(NPW + 1)[None, :]]
  rpw = jnp.pad(rpw, ((0, 0), (0, 336 - (NPW + 1))))

  fold128 = _make_fold(128)
  fold32 = _make_fold(32)

  s0 = fold128(node_feat, srcs_p, fdl_p, rpw)
  cur0 = _layer(s0, node_feat, b0[None], deg, W0)
  s1 = fold32(cur0, srcs_p, fdl_p, rpw)
  cur1 = _layer(s1, cur0, b1[None], deg, W1)
  s2 = fold32(cur1, srcs_p, fdl_p, rpw)
  cur2 = _layer(s2, cur1, b2[None], deg, W2)
  s3 = fold32(cur2, srcs_p, fdl_p, rpw)
  cur3 = _layer(s3, cur2, b3[None], deg, W3)

  sortc = cur3[:, 0].reshape(G, PER)
  spad = jnp.pad(sortc, ((0, 4), (0, 28)), constant_values=-1e30)
  gidx = _topk(spad)
  global_idx = gidx[:G, :K]

  cats = jnp.concatenate(
      [cur0, cur1, cur2, cur3, jnp.zeros((N, 15), jnp.float32)], axis=1)
  pooled = _make_gather_rows()(cats, gidx.reshape(-1))
  pooled3 = pooled.reshape(104, 32, 112)[:G]

  w1eff = jnp.pad(conv1_w[:, 0, :].T, ((0, 15), (0, 0)))
  w2s = jnp.transpose(conv2_w, (2, 1, 0))
  eye = jnp.eye(16, dtype=jnp.float32)
  sel = jnp.stack([
      jnp.kron(eye, jnp.array([[1.0, 0.0]], jnp.float32)),
      jnp.kron(eye, jnp.array([[0.0, 1.0]], jnp.float32)),
  ])

  z = _convs(pooled3, w1eff, conv1_b[None], w2s, conv2_b[None], sel)
  out = z.transpose(0, 2, 1).reshape(G, -1)
  return (out, global_idx)
